# baseline (speedup 1.0000x reference)
import functools

import jax
import jax.numpy as jnp
from jax import lax
from jax.experimental import pallas as pl
from jax.experimental.pallas import tpu as pltpu
from jax.experimental.pallas import tpu_sc as plsc

_NUM_E = 1000000
_BATCH = 16384
_FIELDS = 26
_D = 32
_BF = _BATCH * _FIELDS

_NC = 2
_NS = 16
_NW = _NC * _NS
_PER_W = _BF // _NW
_ILEN = 128
_IDX_ROWS = _PER_W // _ILEN
_SUB = 4
_CHUNK = _SUB * _ILEN
_OUTER = _PER_W // _CHUNK


_BPW = 512

_NBLK = 7813
_MAIN_T = 244


def _transpose_block(src_v, dst_v, iot, n_groups, e_off):
    for gr in range(n_groups):
        for u in range(8):
            f_idx = iot + (16 * (u & 1))
            e_idx = iot * 0 + (4 * gr + (u >> 1) + e_off)
            dst_v[gr, pl.ds(u * 16, 16)] = plsc.load_gather(
                src_v, [f_idx, e_idx]
            )


def _tr_body(tableT_hbm, tailx_hbm, x_hbm, src_v, dst_v):
    wid = lax.axis_index("s") * _NC + lax.axis_index("c")
    iot = lax.iota(jnp.int32, 16)

    def do_block(b):
        off = pl.multiple_of(b * 128, 128)
        pltpu.sync_copy(tableT_hbm.at[pl.ds(0, 32), pl.ds(off, 128)],
                        src_v)
        _transpose_block(src_v, dst_v, iot, 32, 0)
        pltpu.sync_copy(dst_v, x_hbm.at[pl.ds(b * 32, 32)])

    def body(t, carry):
        do_block(t * _NW + wid)
        return carry

    lax.fori_loop(0, _MAIN_T, body, 0)
    @pl.when(wid < 4)
    def _():
        do_block(7808 + wid)

    @pl.when(wid == 4)
    def _():
        pltpu.sync_copy(tailx_hbm, dst_v.at[pl.ds(0, 16)])
        pltpu.sync_copy(dst_v.at[pl.ds(0, 16)], x_hbm.at[pl.ds(249984, 16)])


@functools.partial(
    pl.kernel,
    out_type=jax.ShapeDtypeStruct((250000, 128), jnp.float32),
    mesh=plsc.VectorSubcoreMesh(core_axis_name="c", subcore_axis_name="s"),
    compiler_params=pltpu.CompilerParams(
        use_tc_tiling_on_sc=True, needs_layout_passes=False
    ),
    scratch_types=[
        pltpu.VMEM((32, 128), jnp.float32),
        pltpu.VMEM((32, 128), jnp.float32),
    ],
)
def _sc_transpose(tableT_hbm, tailx_hbm, x_hbm, src_v, dst_v):
    _tr_body(tableT_hbm, tailx_hbm, x_hbm, src_v, dst_v)


def _gather_body(table_hbm, idx_hbm, out_hbm, slab_v, idx_v, rows0, rows1,
                 g0, g1):
    wid = lax.axis_index("s") * _NC + lax.axis_index("c")
    base = wid * _PER_W
    pltpu.sync_copy(idx_hbm.at[pl.ds(wid * _BPW, _BPW)], slab_v)
    lanes = lax.iota(jnp.int32, 16)

    def repack(t, carry):
        for u in range(_ILEN // 16):
            n = t * _ILEN + u * 16 + lanes
            r = (n * 40330) >> 20
            c = n - r * 26
            idx_v[t, pl.ds(u * 16, 16)] = plsc.load_gather(slab_v, [r, c])
        return carry

    lax.fori_loop(0, _IDX_ROWS, repack, 0)

    def fire(t, rows_ref, sem):
        for b in range(_SUB):
            pltpu.async_copy(
                table_hbm.at[idx_v.at[t * _SUB + b]],
                rows_ref.at[pl.ds(b * _ILEN, _ILEN)],
                sem,
            )

    def drain(rows_ref, sem):
        pltpu.make_async_copy(
            table_hbm.at[pl.ds(0, _CHUNK)], rows_ref, sem
        ).wait()

    def writeback(t, rows_ref):
        pltpu.sync_copy(rows_ref, out_hbm.at[pl.ds(base + t * _CHUNK, _CHUNK)])

    fire(0, rows0, g0)
    fire(1, rows1, g1)

    def body(i, carry):
        t = 2 * i
        drain(rows0, g0)
        writeback(t, rows0)
        fire(t + 2, rows0, g0)
        drain(rows1, g1)
        writeback(t + 1, rows1)
        fire(t + 3, rows1, g1)
        return carry

    lax.fori_loop(0, _OUTER // 2 - 1, body, 0)

    drain(rows0, g0)
    writeback(_OUTER - 2, rows0)
    drain(rows1, g1)
    writeback(_OUTER - 1, rows1)


@functools.partial(
    pl.kernel,
    out_type=jax.ShapeDtypeStruct((_BF, _D), jnp.float32),
    mesh=plsc.VectorSubcoreMesh(core_axis_name="c", subcore_axis_name="s"),
    compiler_params=pltpu.CompilerParams(
        use_tc_tiling_on_sc=False, needs_layout_passes=False
    ),
    scratch_types=[
        pltpu.VMEM((_BPW, _FIELDS), jnp.int32),
        pltpu.VMEM((_IDX_ROWS, _ILEN), jnp.int32),
        pltpu.VMEM((_CHUNK, _D), jnp.float32),
        pltpu.VMEM((_CHUNK, _D), jnp.float32),
        pltpu.SemaphoreType.DMA,
        pltpu.SemaphoreType.DMA,
    ],
)
def _sc_gather(table_hbm, idx_hbm, out_hbm, slab_v, idx_v, rows0, rows1,
               g0, g1):
    _gather_body(table_hbm, idx_hbm, out_hbm, slab_v, idx_v, rows0, rows1,
                 g0, g1)


def _proj_body(r_ref, wpt_ref, cr_ref, wpct_ref, zr_ref, zc_ref):
    zr_ref[...] = jnp.dot(r_ref[...], wpt_ref[...],
                          preferred_element_type=jnp.float32)
    zc_ref[...] = jnp.dot(cr_ref[...], wpct_ref[...],
                          preferred_element_type=jnp.float32)


_tc_proj = pl.pallas_call(
    _proj_body,
    out_shape=[
        jax.ShapeDtypeStruct((500, 128), jnp.float32),
        jax.ShapeDtypeStruct((500, 128), jnp.float32),
    ],
)


def kernel(indices, R, C_R, entity_table, W_p, W_p_c):
    tailx = entity_table[999936:].reshape(16, 128)
    table_rm = _sc_transpose(entity_table.T, tailx).reshape(_NUM_E, _D)
    ent_flat = _sc_gather(table_rm, indices.astype(jnp.int32))
    ent = ent_flat.reshape(_BATCH, _FIELDS, _D)
    z_R, z_C = _tc_proj(R, W_p.T, C_R.astype(R.dtype), W_p_c.T)
    return ent, z_R, z_C


# speedup vs baseline: 1.4969x; 1.4969x over previous
import functools

import jax
import jax.numpy as jnp
from jax import lax
from jax.experimental import pallas as pl
from jax.experimental.pallas import tpu as pltpu
from jax.experimental.pallas import tpu_sc as plsc

_NUM_E = 1000000
_BATCH = 16384
_FIELDS = 26
_D = 32
_BF = _BATCH * _FIELDS

_NC = 2
_NS = 16
_NW = _NC * _NS
_PER_W = _BF // _NW
_ILEN = 128
_IDX_ROWS = _PER_W // _ILEN
_SUB = 4
_CHUNK = _SUB * _ILEN
_OUTER = _PER_W // _CHUNK


_BPW = 512

_NBLK = 7813
_MAIN_T = 244


def _transpose_block(src_v, dst_v, rows_t, colb):
    for f in range(32):
        cv = colb + f
        for t in range(8):
            plsc.store_scatter(dst_v, [rows_t[t], cv],
                               src_v[f, pl.ds(16 * t, 16)])


def _tr_body(tableT_hbm, tailx_hbm, x_hbm, s0, s1, d0, d1, ls0, ls1,
             ws0, ws1):
    wid = lax.axis_index("s") * _NC + lax.axis_index("c")
    iot = lax.iota(jnp.int32, 16)
    rows_t = [(iot >> 2) + 4 * t for t in range(8)]
    colb = (iot & 3) * 32

    def load(t, s, sem):
        off = pl.multiple_of((t * _NW + wid) * 128, 128)
        pltpu.async_copy(tableT_hbm.at[pl.ds(0, 32), pl.ds(off, 128)], s,
                         sem)

    def drain_load(s, sem):
        pltpu.make_async_copy(
            tableT_hbm.at[pl.ds(0, 32), pl.ds(0, 128)], s, sem
        ).wait()

    def wb(t, d, sem):
        pltpu.async_copy(d, x_hbm.at[pl.ds((t * _NW + wid) * 32, 32)], sem)

    def drain_wb(d, sem):
        pltpu.make_async_copy(d, x_hbm.at[pl.ds(0, 32)], sem).wait()

    load(0, s0, ls0)
    load(1, s1, ls1)

    def body(i, carry):
        for (t, s, d, ls, ws) in ((2 * i, s0, d0, ls0, ws0),
                                  (2 * i + 1, s1, d1, ls1, ws1)):
            drain_load(s, ls)

            @pl.when(i > 0)
            def _():
                drain_wb(d, ws)

            _transpose_block(s, d, rows_t, colb)
            wb(t, d, ws)

            @pl.when(i < _MAIN_T // 2 - 1)
            def _():
                load(t + 2, s, ls)
        return carry

    lax.fori_loop(0, _MAIN_T // 2, body, 0)
    drain_wb(d0, ws0)
    drain_wb(d1, ws1)
    @pl.when(wid < 4)
    def _():
        b = 7808 + wid
        off = pl.multiple_of(b * 128, 128)
        pltpu.sync_copy(tableT_hbm.at[pl.ds(0, 32), pl.ds(off, 128)], s0)
        _transpose_block(s0, d0, rows_t, colb)
        pltpu.sync_copy(d0, x_hbm.at[pl.ds(b * 32, 32)])

    @pl.when(wid == 4)
    def _():
        pltpu.sync_copy(tailx_hbm, d1.at[pl.ds(0, 16)])
        pltpu.sync_copy(d1.at[pl.ds(0, 16)], x_hbm.at[pl.ds(249984, 16)])


@functools.partial(
    pl.kernel,
    out_type=jax.ShapeDtypeStruct((250000, 128), jnp.float32),
    mesh=plsc.VectorSubcoreMesh(core_axis_name="c", subcore_axis_name="s"),
    compiler_params=pltpu.CompilerParams(
        use_tc_tiling_on_sc=True, needs_layout_passes=False
    ),
    scratch_types=[
        pltpu.VMEM((32, 128), jnp.float32),
        pltpu.VMEM((32, 128), jnp.float32),
        pltpu.VMEM((32, 128), jnp.float32),
        pltpu.VMEM((32, 128), jnp.float32),
        pltpu.SemaphoreType.DMA,
        pltpu.SemaphoreType.DMA,
        pltpu.SemaphoreType.DMA,
        pltpu.SemaphoreType.DMA,
    ],
)
def _sc_transpose(tableT_hbm, tailx_hbm, x_hbm, s0, s1, d0, d1, ls0, ls1,
                  ws0, ws1):
    _tr_body(tableT_hbm, tailx_hbm, x_hbm, s0, s1, d0, d1, ls0, ls1,
             ws0, ws1)


def _gather_body(table_hbm, idx_hbm, out_hbm, slab_v, idx_v, rows0, rows1,
                 g0, g1):
    wid = lax.axis_index("s") * _NC + lax.axis_index("c")
    base = wid * _PER_W
    pltpu.sync_copy(idx_hbm.at[pl.ds(wid * _BPW, _BPW)], slab_v)
    lanes = lax.iota(jnp.int32, 16)

    def repack(t, carry):
        for u in range(_ILEN // 16):
            n = t * _ILEN + u * 16 + lanes
            r = (n * 40330) >> 20
            c = n - r * 26
            idx_v[t, pl.ds(u * 16, 16)] = plsc.load_gather(slab_v, [r, c])
        return carry

    lax.fori_loop(0, _IDX_ROWS, repack, 0)

    def fire(t, rows_ref, sem):
        for b in range(_SUB):
            pltpu.async_copy(
                table_hbm.at[idx_v.at[t * _SUB + b]],
                rows_ref.at[pl.ds(b * _ILEN, _ILEN)],
                sem,
            )

    def drain(rows_ref, sem):
        pltpu.make_async_copy(
            table_hbm.at[pl.ds(0, _CHUNK)], rows_ref, sem
        ).wait()

    def writeback(t, rows_ref):
        pltpu.sync_copy(rows_ref, out_hbm.at[pl.ds(base + t * _CHUNK, _CHUNK)])

    fire(0, rows0, g0)
    fire(1, rows1, g1)

    def body(i, carry):
        t = 2 * i
        drain(rows0, g0)
        writeback(t, rows0)
        fire(t + 2, rows0, g0)
        drain(rows1, g1)
        writeback(t + 1, rows1)
        fire(t + 3, rows1, g1)
        return carry

    lax.fori_loop(0, _OUTER // 2 - 1, body, 0)

    drain(rows0, g0)
    writeback(_OUTER - 2, rows0)
    drain(rows1, g1)
    writeback(_OUTER - 1, rows1)


@functools.partial(
    pl.kernel,
    out_type=jax.ShapeDtypeStruct((_BF, _D), jnp.float32),
    mesh=plsc.VectorSubcoreMesh(core_axis_name="c", subcore_axis_name="s"),
    compiler_params=pltpu.CompilerParams(
        use_tc_tiling_on_sc=False, needs_layout_passes=False
    ),
    scratch_types=[
        pltpu.VMEM((_BPW, _FIELDS), jnp.int32),
        pltpu.VMEM((_IDX_ROWS, _ILEN), jnp.int32),
        pltpu.VMEM((_CHUNK, _D), jnp.float32),
        pltpu.VMEM((_CHUNK, _D), jnp.float32),
        pltpu.SemaphoreType.DMA,
        pltpu.SemaphoreType.DMA,
    ],
)
def _sc_gather(table_hbm, idx_hbm, out_hbm, slab_v, idx_v, rows0, rows1,
               g0, g1):
    _gather_body(table_hbm, idx_hbm, out_hbm, slab_v, idx_v, rows0, rows1,
                 g0, g1)


def _proj_body(r_ref, wpt_ref, cr_ref, wpct_ref, zr_ref, zc_ref):
    zr_ref[...] = jnp.dot(r_ref[...], wpt_ref[...],
                          preferred_element_type=jnp.float32)
    zc_ref[...] = jnp.dot(cr_ref[...], wpct_ref[...],
                          preferred_element_type=jnp.float32)


_tc_proj = pl.pallas_call(
    _proj_body,
    out_shape=[
        jax.ShapeDtypeStruct((500, 128), jnp.float32),
        jax.ShapeDtypeStruct((500, 128), jnp.float32),
    ],
)


def kernel(indices, R, C_R, entity_table, W_p, W_p_c):
    tailx = entity_table[999936:].reshape(16, 128)
    table_rm = _sc_transpose(entity_table.T, tailx).reshape(_NUM_E, _D)
    ent_flat = _sc_gather(table_rm, indices.astype(jnp.int32))
    ent = ent_flat.reshape(_BATCH, _FIELDS, _D)
    z_R, z_C = _tc_proj(R, W_p.T, C_R.astype(R.dtype), W_p_c.T)
    return ent, z_R, z_C


# speedup vs baseline: 1.8619x; 1.2438x over previous
import functools

import jax
import jax.numpy as jnp
from jax import lax
from jax.experimental import pallas as pl
from jax.experimental.pallas import tpu as pltpu
from jax.experimental.pallas import tpu_sc as plsc

_BATCH = 16384
_FIELDS = 26
_D = 32
_BF = _BATCH * _FIELDS

_NC = 2
_NS = 16
_NW = _NC * _NS
_PER_W = _BF // _NW
_ILEN = 128
_IDX_ROWS = _PER_W // _ILEN
_SUB = 4
_CHUNK = _SUB * _ILEN
_OUTER = _PER_W // _CHUNK


def _gather_body(table_hbm, idx_hbm, out_hbm, idx_v, rows0, rows1, g0, g1,
                 isem):
    wid = lax.axis_index("s") * _NC + lax.axis_index("c")
    base = wid * _PER_W
    g_base = wid * _IDX_ROWS
    cps = []
    for k in range(_IDX_ROWS):
        g = g_base + k
        f = g // 128
        c = g % 128
        cps.append(
            pltpu.async_copy(
                idx_hbm.at[f, pl.ds(c * _ILEN, _ILEN)], idx_v.at[k], isem
            )
        )
    for cp in cps:
        cp.wait()

    def fire(t, rows_ref, sem):
        for b in range(_SUB):
            pltpu.async_copy(
                table_hbm.at[idx_v.at[t * _SUB + b]],
                rows_ref.at[pl.ds(b * _ILEN, _ILEN)],
                sem,
            )

    def drain(rows_ref, sem):
        pltpu.make_async_copy(
            table_hbm.at[pl.ds(0, _CHUNK)], rows_ref, sem
        ).wait()

    def writeback(t, rows_ref):
        pltpu.sync_copy(rows_ref, out_hbm.at[pl.ds(base + t * _CHUNK, _CHUNK)])

    fire(0, rows0, g0)
    fire(1, rows1, g1)

    def body(i, carry):
        t = 2 * i
        drain(rows0, g0)
        writeback(t, rows0)
        fire(t + 2, rows0, g0)
        drain(rows1, g1)
        writeback(t + 1, rows1)
        fire(t + 3, rows1, g1)
        return carry

    lax.fori_loop(0, _OUTER // 2 - 1, body, 0)

    drain(rows0, g0)
    writeback(_OUTER - 2, rows0)
    drain(rows1, g1)
    writeback(_OUTER - 1, rows1)


@functools.partial(
    pl.kernel,
    out_type=jax.ShapeDtypeStruct((_BF, _D), jnp.float32),
    mesh=plsc.VectorSubcoreMesh(core_axis_name="c", subcore_axis_name="s"),
    compiler_params=pltpu.CompilerParams(use_tc_tiling_on_sc=False),
    scratch_types=[
        pltpu.VMEM((_IDX_ROWS, _ILEN), jnp.int32),
        pltpu.VMEM((_CHUNK, _D), jnp.float32),
        pltpu.VMEM((_CHUNK, _D), jnp.float32),
        pltpu.SemaphoreType.DMA,
        pltpu.SemaphoreType.DMA,
        pltpu.SemaphoreType.DMA,
    ],
)
def _sc_gather(table_hbm, idx_hbm, out_hbm, idx_v, rows0, rows1, g0, g1,
               isem):
    _gather_body(table_hbm, idx_hbm, out_hbm, idx_v, rows0, rows1, g0, g1,
                 isem)


def _proj_body(r_ref, wpt_ref, cr_ref, wpct_ref, zr_ref, zc_ref):
    zr_ref[...] = jnp.dot(r_ref[...], wpt_ref[...],
                          preferred_element_type=jnp.float32)
    zc_ref[...] = jnp.dot(cr_ref[...], wpct_ref[...],
                          preferred_element_type=jnp.float32)


_tc_proj = pl.pallas_call(
    _proj_body,
    out_shape=[
        jax.ShapeDtypeStruct((500, 128), jnp.float32),
        jax.ShapeDtypeStruct((500, 128), jnp.float32),
    ],
)


def kernel(indices, R, C_R, entity_table, W_p, W_p_c):
    ent_flat = _sc_gather(entity_table, indices.T.astype(jnp.int32))
    ent = ent_flat.reshape(_FIELDS, _BATCH, _D).transpose(1, 0, 2)
    z_R, z_C = _tc_proj(R, W_p.T, C_R.astype(R.dtype), W_p_c.T)
    return ent, z_R, z_C
